# Initial kernel scaffold; baseline (speedup 1.0000x reference)
import jax
import jax.numpy as jnp
from jax import lax
from jax.experimental import pallas as pl
from jax.experimental.pallas import tpu as pltpu
from jax.experimental.pallas import tpu_sc as plsc

NUM_USERS = 1000000
NUM_SUBS = 100000
D = 32
B = 16384
K = 20

NC = 2
NS = 16
NW = NC * NS
BPW = B // NW
CHUNK = 128
NCHUNK = BPW // CHUNK
GATHER_N = 128
NEG_PER_CHUNK = CHUNK * K
NEG_GATHERS = NEG_PER_CHUNK // GATHER_N
GROUPS = CHUNK // 16


def _sc_body(uid_hbm, pid_hbm, nid_hbm, user_emb, sub_emb,
             pos_out, neg_out,
             idx_u, idx_p, idx_n, u_v, sp_v, sn_v, pos_v, neg_v, sem):
    wid = lax.axis_index("s") * NC + lax.axis_index("c")

    def chunk_body(c, carry):
        base = wid * BPW + c * CHUNK
        nrow0 = (wid * NCHUNK + c) * NEG_GATHERS

        pltpu.sync_copy(uid_hbm.at[pl.ds(base, CHUNK)], idx_u)
        pltpu.sync_copy(pid_hbm.at[pl.ds(base, CHUNK)], idx_p)
        pltpu.sync_copy(nid_hbm.at[pl.ds(nrow0, NEG_GATHERS), :], idx_n)

        cps = [pltpu.async_copy(user_emb.at[idx_u], u_v, sem),
               pltpu.async_copy(sub_emb.at[idx_p], sp_v, sem)]
        for j in range(NEG_GATHERS):
            cps.append(pltpu.async_copy(
                sub_emb.at[idx_n.at[j]],
                sn_v.at[pl.ds(j * GATHER_N, GATHER_N), :], sem))
        for cp in cps:
            cp.wait()

        def group_body(g, gcarry):
            rows = g * 16 + lax.iota(jnp.int32, 16)
            rows_k = rows * K
            zero = jnp.zeros((16,), jnp.float32)

            def d_body(d, accs):
                cold = jnp.full((16,), d, jnp.int32)
                u_d = plsc.load_gather(u_v, [rows, cold])
                p_d = plsc.load_gather(sp_v, [rows, cold])
                out = [accs[0] + u_d * p_d]
                for k in range(K):
                    n_d = plsc.load_gather(sn_v, [rows_k + k, cold])
                    out.append(accs[k + 1] + u_d * n_d)
                return tuple(out)

            accs = lax.fori_loop(0, D, d_body, (zero,) * (K + 1))
            pos_v[pl.ds(g * 16, 16)] = accs[0]
            for k in range(K):
                plsc.store_scatter(neg_v, [rows_k + k], accs[k + 1])
            return gcarry

        lax.fori_loop(0, GROUPS, group_body, 0)

        pltpu.sync_copy(pos_v, pos_out.at[pl.ds(base, CHUNK)])
        pltpu.sync_copy(neg_v, neg_out.at[pl.ds(base * K, NEG_PER_CHUNK)])
        return carry

    lax.fori_loop(0, NCHUNK, chunk_body, 0)


@jax.jit
def _sc_forward(user_ids, pos_sub_ids, neg_ids_2d, user_emb, sub_emb):
    mesh = plsc.VectorSubcoreMesh(core_axis_name="c", subcore_axis_name="s")
    return pl.kernel(
        _sc_body,
        out_type=(jax.ShapeDtypeStruct((B,), jnp.float32),
                  jax.ShapeDtypeStruct((B * K,), jnp.float32)),
        mesh=mesh,
        scratch_types=[
            pltpu.VMEM((CHUNK,), jnp.int32),
            pltpu.VMEM((CHUNK,), jnp.int32),
            pltpu.VMEM((NEG_GATHERS, GATHER_N), jnp.int32),
            pltpu.VMEM((CHUNK, D), jnp.float32),
            pltpu.VMEM((CHUNK, D), jnp.float32),
            pltpu.VMEM((NEG_PER_CHUNK, D), jnp.float32),
            pltpu.VMEM((CHUNK,), jnp.float32),
            pltpu.VMEM((NEG_PER_CHUNK,), jnp.float32),
            pltpu.SemaphoreType.DMA,
        ],
    )(user_ids, pos_sub_ids, neg_ids_2d, user_emb, sub_emb)


def kernel(user_ids, pos_sub_ids, neg_sub_ids, user_emb, sub_emb):
    uid = user_ids.astype(jnp.int32)
    pid = pos_sub_ids.astype(jnp.int32)
    nid = neg_sub_ids.astype(jnp.int32).reshape(B * K // GATHER_N, GATHER_N)
    pos_flat, neg_flat = _sc_forward(uid, pid, nid, user_emb, sub_emb)
    return (pos_flat, neg_flat.reshape(B, K))


# speedup vs baseline: 2.1912x; 2.1912x over previous
import jax
import jax.numpy as jnp
from jax import lax
from jax.experimental import pallas as pl
from jax.experimental.pallas import tpu as pltpu
from jax.experimental.pallas import tpu_sc as plsc

NUM_USERS = 1000000
NUM_SUBS = 100000
D = 32
B = 16384
K = 20

NC = 2
NS = 16
NW = NC * NS
BPW = B // NW
CHUNK = 128
NCHUNK = BPW // CHUNK
GATHER_N = 128
NEG_PER_CHUNK = CHUNK * K
NEG_GATHERS = NEG_PER_CHUNK // GATHER_N
GROUPS = CHUNK // 16


def _sc_body(uid_hbm, pid_hbm, nid_hbm, user_emb, sub_emb,
             pos_out, neg_out,
             idx_u, idx_p, idx_n, u_v, sp_v, sn_v, pos_v, neg_v, sem):
    wid = lax.axis_index("s") * NC + lax.axis_index("c")

    def chunk_body(c, carry):
        base = wid * BPW + c * CHUNK

        pltpu.sync_copy(uid_hbm.at[pl.ds(base, CHUNK)], idx_u)
        pltpu.sync_copy(pid_hbm.at[pl.ds(base, CHUNK)], idx_p)
        pltpu.sync_copy(nid_hbm.at[pl.ds(base * K, NEG_PER_CHUNK)], idx_n)

        cps = [pltpu.async_copy(user_emb.at[idx_u], u_v, sem),
               pltpu.async_copy(sub_emb.at[idx_p], sp_v, sem)]
        for j in range(NEG_GATHERS):
            cps.append(pltpu.async_copy(
                sub_emb.at[idx_n.at[pl.ds(j * GATHER_N, GATHER_N)]],
                sn_v.at[pl.ds(j * GATHER_N, GATHER_N), :], sem))
        for cp in cps:
            cp.wait()

        def group_body(g, gcarry):
            rows = g * 16 + lax.iota(jnp.int32, 16)
            rows_k = rows * K
            zero = jnp.zeros((16,), jnp.float32)

            def d_body(d, accs):
                cold = jnp.full((16,), d, jnp.int32)
                u_d = plsc.load_gather(u_v, [rows, cold])
                p_d = plsc.load_gather(sp_v, [rows, cold])
                out = [accs[0] + u_d * p_d]
                for k in range(K):
                    n_d = plsc.load_gather(sn_v, [rows_k + k, cold])
                    out.append(accs[k + 1] + u_d * n_d)
                return tuple(out)

            accs = lax.fori_loop(0, D, d_body, (zero,) * (K + 1))
            pos_v[pl.ds(g * 16, 16)] = accs[0]
            for k in range(K):
                plsc.store_scatter(neg_v, [rows_k + k], accs[k + 1])
            return gcarry

        lax.fori_loop(0, GROUPS, group_body, 0)

        pltpu.sync_copy(pos_v, pos_out.at[pl.ds(base, CHUNK)])
        pltpu.sync_copy(neg_v, neg_out.at[pl.ds(base * K, NEG_PER_CHUNK)])
        return carry

    lax.fori_loop(0, NCHUNK, chunk_body, 0)


@jax.jit
def _sc_forward(user_ids, pos_sub_ids, neg_ids_2d, user_emb, sub_emb):
    mesh = plsc.VectorSubcoreMesh(core_axis_name="c", subcore_axis_name="s")
    return pl.kernel(
        _sc_body,
        out_type=(jax.ShapeDtypeStruct((B,), jnp.float32),
                  jax.ShapeDtypeStruct((B * K,), jnp.float32)),
        mesh=mesh,
        scratch_types=[
            pltpu.VMEM((CHUNK,), jnp.int32),
            pltpu.VMEM((CHUNK,), jnp.int32),
            pltpu.VMEM((NEG_PER_CHUNK,), jnp.int32),
            pltpu.VMEM((CHUNK, D), jnp.float32),
            pltpu.VMEM((CHUNK, D), jnp.float32),
            pltpu.VMEM((NEG_PER_CHUNK, D), jnp.float32),
            pltpu.VMEM((CHUNK,), jnp.float32),
            pltpu.VMEM((NEG_PER_CHUNK,), jnp.float32),
            pltpu.SemaphoreType.DMA,
        ],
        compiler_params=pltpu.CompilerParams(use_tc_tiling_on_sc=False, needs_layout_passes=False),
    )(user_ids, pos_sub_ids, neg_ids_2d, user_emb, sub_emb)


def kernel(user_ids, pos_sub_ids, neg_sub_ids, user_emb, sub_emb):
    uid = user_ids.astype(jnp.int32)
    pid = pos_sub_ids.astype(jnp.int32)
    nid = neg_sub_ids.astype(jnp.int32).reshape(B * K)
    pos_flat, neg_flat = _sc_forward(uid, pid, nid, user_emb, sub_emb)
    return (pos_flat, neg_flat.reshape(B, K))


# speedup vs baseline: 2.8815x; 1.3150x over previous
import jax
import jax.numpy as jnp
from jax import lax
from jax.experimental import pallas as pl
from jax.experimental.pallas import tpu as pltpu
from jax.experimental.pallas import tpu_sc as plsc

NUM_USERS = 1000000
NUM_SUBS = 100000
D = 32
B = 16384
K = 20

NC = 2
NS = 16
NW = NC * NS
BPW = B // NW
CHUNK = 128
NCHUNK = BPW // CHUNK
GATHER_N = 128
NEG_PER_CHUNK = CHUNK * K
NEG_GATHERS = NEG_PER_CHUNK // GATHER_N
GROUPS = CHUNK // 16


def _sc_body(uid_hbm, pid_hbm, nid_hbm, user_emb, sub_emb,
             pos_out, neg_out,
             idx_u, idx_p, idx_n, u_v, sp_v, sn_v, pos_v, neg_v, sem):
    wid = lax.axis_index("s") * NC + lax.axis_index("c")

    def chunk_body(c, carry):
        base = wid * BPW + c * CHUNK

        pltpu.sync_copy(uid_hbm.at[pl.ds(base, CHUNK)], idx_u)
        pltpu.sync_copy(pid_hbm.at[pl.ds(base, CHUNK)], idx_p)
        pltpu.sync_copy(nid_hbm.at[pl.ds(base * K, NEG_PER_CHUNK)], idx_n)

        cps = [pltpu.async_copy(user_emb.at[idx_u], u_v, sem),
               pltpu.async_copy(sub_emb.at[idx_p], sp_v, sem)]
        for j in range(NEG_GATHERS):
            cps.append(pltpu.async_copy(
                sub_emb.at[idx_n.at[pl.ds(j * GATHER_N, GATHER_N)]],
                sn_v.at[pl.ds(j * GATHER_N, GATHER_N), :], sem))
        for cp in cps:
            cp.wait()

        def group_body(g, gcarry):
            rows = g * 16 + lax.iota(jnp.int32, 16)
            rows_k = rows * K
            zero = jnp.zeros((16,), jnp.float32)

            lanes = lax.iota(jnp.int32, 16)

            def d_body(d, accs):
                cold = (lanes + d) & (D - 1)
                u_d = plsc.load_gather(u_v, [rows, cold])
                p_d = plsc.load_gather(sp_v, [rows, cold])
                out = [accs[0] + u_d * p_d]
                for k in range(K):
                    n_d = plsc.load_gather(sn_v, [rows_k + k, cold])
                    out.append(accs[k + 1] + u_d * n_d)
                return tuple(out)

            accs = lax.fori_loop(0, D, d_body, (zero,) * (K + 1))
            pos_v[pl.ds(g * 16, 16)] = accs[0]
            for k in range(K):
                plsc.store_scatter(neg_v, [rows_k + k], accs[k + 1])
            return gcarry

        lax.fori_loop(0, GROUPS, group_body, 0)

        pltpu.sync_copy(pos_v, pos_out.at[pl.ds(base, CHUNK)])
        pltpu.sync_copy(neg_v, neg_out.at[pl.ds(base * K, NEG_PER_CHUNK)])
        return carry

    lax.fori_loop(0, NCHUNK, chunk_body, 0)


@jax.jit
def _sc_forward(user_ids, pos_sub_ids, neg_ids_2d, user_emb, sub_emb):
    mesh = plsc.VectorSubcoreMesh(core_axis_name="c", subcore_axis_name="s")
    return pl.kernel(
        _sc_body,
        out_type=(jax.ShapeDtypeStruct((B,), jnp.float32),
                  jax.ShapeDtypeStruct((B * K,), jnp.float32)),
        mesh=mesh,
        scratch_types=[
            pltpu.VMEM((CHUNK,), jnp.int32),
            pltpu.VMEM((CHUNK,), jnp.int32),
            pltpu.VMEM((NEG_PER_CHUNK,), jnp.int32),
            pltpu.VMEM((CHUNK, D), jnp.float32),
            pltpu.VMEM((CHUNK, D), jnp.float32),
            pltpu.VMEM((NEG_PER_CHUNK, D), jnp.float32),
            pltpu.VMEM((CHUNK,), jnp.float32),
            pltpu.VMEM((NEG_PER_CHUNK,), jnp.float32),
            pltpu.SemaphoreType.DMA,
        ],
        compiler_params=pltpu.CompilerParams(use_tc_tiling_on_sc=False, needs_layout_passes=False),
    )(user_ids, pos_sub_ids, neg_ids_2d, user_emb, sub_emb)


def kernel(user_ids, pos_sub_ids, neg_sub_ids, user_emb, sub_emb):
    uid = user_ids.astype(jnp.int32)
    pid = pos_sub_ids.astype(jnp.int32)
    nid = neg_sub_ids.astype(jnp.int32).reshape(B * K)
    pos_flat, neg_flat = _sc_forward(uid, pid, nid, user_emb, sub_emb)
    return (pos_flat, neg_flat.reshape(B, K))
